# baseline (speedup 1.0000x reference)
import functools

import jax
import jax.numpy as jnp
from jax.experimental import pallas as pl
from jax.experimental.pallas import tpu as pltpu

_TAU = 0.2
_K = 15
_KP = 16
_CW = 64


def _normalize_kernel(x_ref, o_ref):
    x = x_ref[0]
    n = jnp.sqrt(jnp.sum(x * x, axis=-1, keepdims=True))
    o_ref[0] = x / jnp.maximum(n, 1e-12)


def _l2norm(x):
    b, n, c = x.shape
    return pl.pallas_call(
        _normalize_kernel,
        grid=(b,),
        in_specs=[pl.BlockSpec((1, n, c), lambda i: (i, 0, 0))],
        out_specs=pl.BlockSpec((1, n, c), lambda i: (i, 0, 0)),
        out_shape=jax.ShapeDtypeStruct((b, n, c), jnp.float32),
    )(x)


def _fused_topk_kernel(fx_ref, fyn_ref, idx_ref, val_ref, *, ny, k):
    fx = fx_ref[0]
    fyn = fyn_ref[0]

    nx = jnp.sqrt(jnp.sum(fx * fx, axis=-1, keepdims=True))
    fxn = fx / jnp.maximum(nx, 1e-12)

    simt = jax.lax.dot_general(
        fyn, fxn, (((1,), (1,)), ((), ())),
        preferred_element_type=jnp.float32,
    )

    r = simt.shape[1]
    cw = _CW
    nch = ny // cw
    neg = jnp.finfo(jnp.float32).min
    big = jnp.int32(1 << 30)

    iota_ch = jax.lax.broadcasted_iota(jnp.int32, (nch, r), 0)
    iota_w = jax.lax.broadcasted_iota(jnp.int32, (cw, r), 0)

    slices = [simt[c * cw:(c + 1) * cw, :] for c in range(nch)]

    cms = []
    jms = []
    for s in slices:
        mx = jnp.max(s, axis=0, keepdims=True)
        cms.append(mx)
        jms.append(jnp.min(jnp.where(s == mx, iota_w, cw),
                           axis=0, keepdims=True))
    cm = jnp.concatenate(cms, axis=0)
    jm = jnp.concatenate(jms, axis=0)

    vals = []
    idxs = []
    for t in range(k):
        m = jnp.max(cm, axis=0, keepdims=True)
        g = jnp.min(jnp.where(cm == m, iota_ch * cw + jm, big),
                    axis=0, keepdims=True)
        vals.append(m)
        idxs.append(g)
        if t == k - 1:
            break
        c = jax.lax.shift_right_logical(g, cw.bit_length() - 1)
        p = jax.lax.bitwise_and(g, cw - 1)
        colv = slices[0]
        for j in range(1, nch):
            colv = jnp.where(c == j, slices[j], colv)
        keep = (colv < m) | ((colv == m) & (iota_w > p))
        rem = jnp.where(keep, colv, neg)
        nm = jnp.max(rem, axis=0, keepdims=True)
        npos = jnp.min(jnp.where(rem == nm, iota_w, cw),
                       axis=0, keepdims=True)
        cm = jnp.where(iota_ch == c, nm, cm)
        jm = jnp.where(iota_ch == c, npos, jm)

    vals.append(jnp.full((1, r), neg, jnp.float32))
    idxs.append(jnp.zeros((1, r), jnp.int32))
    v = jnp.concatenate(vals, axis=0)
    i = jnp.concatenate(idxs, axis=0)

    vt = v / jnp.float32(_TAU)
    e = jnp.exp(vt - vt[:1, :])
    sm = e / jnp.sum(e[:_K, :], axis=0, keepdims=True)

    idx_ref[0] = i
    val_ref[0] = sm


def kernel(feat_x, feat_y):
    b, nx, c = feat_x.shape
    ny = feat_y.shape[1]
    r = 512
    grid = (b, nx // r)

    fyn = _l2norm(feat_y)
    body = functools.partial(_fused_topk_kernel, ny=ny, k=_K)

    idx, val = pl.pallas_call(
        body,
        grid=grid,
        in_specs=[
            pl.BlockSpec((1, r, c), lambda bi, i: (bi, i, 0)),
            pl.BlockSpec((1, ny, c), lambda bi, i: (bi, 0, 0)),
        ],
        out_specs=[
            pl.BlockSpec((1, _KP, r), lambda bi, i: (bi, 0, i)),
            pl.BlockSpec((1, _KP, r), lambda bi, i: (bi, 0, i)),
        ],
        out_shape=[
            jax.ShapeDtypeStruct((b, _KP, nx), jnp.int32),
            jax.ShapeDtypeStruct((b, _KP, nx), jnp.float32),
        ],
        compiler_params=pltpu.CompilerParams(
            dimension_semantics=("arbitrary", "arbitrary"),
        ),
    )(feat_x, fyn)
    idx = jnp.transpose(idx[:, :_K, :], (0, 2, 1))
    val = jnp.transpose(val[:, :_K, :], (0, 2, 1))
    return idx, val


# speedup vs baseline: 1.0257x; 1.0257x over previous
import functools

import jax
import jax.numpy as jnp
from jax.experimental import pallas as pl
from jax.experimental.pallas import tpu as pltpu

_TAU = 0.2
_K = 15
_KP = 16
_CW = 64


def _normalize_kernel(x_ref, o_ref):
    x = x_ref[0]
    n = jnp.sqrt(jnp.sum(x * x, axis=-1, keepdims=True))
    o_ref[0] = x / jnp.maximum(n, 1e-12)


def _l2norm(x):
    b, n, c = x.shape
    return pl.pallas_call(
        _normalize_kernel,
        grid=(b,),
        in_specs=[pl.BlockSpec((1, n, c), lambda i: (i, 0, 0))],
        out_specs=pl.BlockSpec((1, n, c), lambda i: (i, 0, 0)),
        out_shape=jax.ShapeDtypeStruct((b, n, c), jnp.float32),
    )(x)


def _fused_topk_kernel(fx_ref, fyn_ref, idx_ref, val_ref, *, ny, k):
    fx = fx_ref[0]
    fyn = fyn_ref[0]

    nx = jnp.sqrt(jnp.sum(fx * fx, axis=-1, keepdims=True))
    fxn = fx / jnp.maximum(nx, 1e-12)

    simt = jax.lax.dot_general(
        fyn, fxn, (((1,), (1,)), ((), ())),
        preferred_element_type=jnp.float32,
    )

    r = simt.shape[1]
    cw = _CW
    nch = ny // cw
    neg = jnp.finfo(jnp.float32).min
    big = jnp.int32(1 << 30)

    iota_ch = jax.lax.broadcasted_iota(jnp.int32, (nch, r), 0)
    iota_w = jax.lax.broadcasted_iota(jnp.int32, (cw, r), 0)

    slices = [simt[c * cw:(c + 1) * cw, :] for c in range(nch)]

    cms = []
    jms = []
    for s in slices:
        mx = jnp.max(s, axis=0, keepdims=True)
        cms.append(mx)
        jms.append(jnp.min(jnp.where(s == mx, iota_w, cw),
                           axis=0, keepdims=True))
    cm = jnp.concatenate(cms, axis=0)
    jm = jnp.concatenate(jms, axis=0)

    vals = []
    idxs = []
    for t in range(k):
        m = jnp.max(cm, axis=0, keepdims=True)
        g = jnp.min(jnp.where(cm == m, iota_ch * cw + jm, big),
                    axis=0, keepdims=True)
        vals.append(m)
        idxs.append(g)
        if t == k - 1:
            break
        c = jax.lax.shift_right_logical(g, cw.bit_length() - 1)
        p = jax.lax.bitwise_and(g, cw - 1)
        colv = slices[0]
        for j in range(1, nch):
            colv = jnp.where(c == j, slices[j], colv)
        keep = (colv < m) | ((colv == m) & (iota_w > p))
        rem = jnp.where(keep, colv, neg)
        nm = jnp.max(rem, axis=0, keepdims=True)
        npos = jnp.min(jnp.where(rem == nm, iota_w, cw),
                       axis=0, keepdims=True)
        cm = jnp.where(iota_ch == c, nm, cm)
        jm = jnp.where(iota_ch == c, npos, jm)

    vals.append(jnp.full((1, r), neg, jnp.float32))
    idxs.append(jnp.zeros((1, r), jnp.int32))
    v = jnp.concatenate(vals, axis=0)
    i = jnp.concatenate(idxs, axis=0)

    vt = v / jnp.float32(_TAU)
    e = jnp.exp(vt - vt[:1, :])
    sm = e / jnp.sum(e[:_K, :], axis=0, keepdims=True)

    idx_ref[0] = i
    val_ref[0] = sm


def kernel(feat_x, feat_y):
    b, nx, c = feat_x.shape
    ny = feat_y.shape[1]
    r = 256
    grid = (b, nx // r)

    fyn = _l2norm(feat_y)
    body = functools.partial(_fused_topk_kernel, ny=ny, k=_K)

    idx, val = pl.pallas_call(
        body,
        grid=grid,
        in_specs=[
            pl.BlockSpec((1, r, c), lambda bi, i: (bi, i, 0)),
            pl.BlockSpec((1, ny, c), lambda bi, i: (bi, 0, 0)),
        ],
        out_specs=[
            pl.BlockSpec((1, _KP, r), lambda bi, i: (bi, 0, i)),
            pl.BlockSpec((1, _KP, r), lambda bi, i: (bi, 0, i)),
        ],
        out_shape=[
            jax.ShapeDtypeStruct((b, _KP, nx), jnp.int32),
            jax.ShapeDtypeStruct((b, _KP, nx), jnp.float32),
        ],
        compiler_params=pltpu.CompilerParams(
            dimension_semantics=("arbitrary", "arbitrary"),
        ),
    )(feat_x, fyn)
    idx = jnp.transpose(idx[:, :_K, :], (0, 2, 1))
    val = jnp.transpose(val[:, :_K, :], (0, 2, 1))
    return idx, val


# speedup vs baseline: 1.0267x; 1.0009x over previous
import functools

import jax
import jax.numpy as jnp
from jax.experimental import pallas as pl
from jax.experimental.pallas import tpu as pltpu

_TAU = 0.2
_K = 15
_KP = 16
_CW = 64


def _normalize_kernel(x_ref, o_ref):
    x = x_ref[0]
    n = jnp.sqrt(jnp.sum(x * x, axis=-1, keepdims=True))
    o_ref[0] = x / jnp.maximum(n, 1e-12)


def _l2norm(x):
    b, n, c = x.shape
    return pl.pallas_call(
        _normalize_kernel,
        grid=(b,),
        in_specs=[pl.BlockSpec((1, n, c), lambda i: (i, 0, 0))],
        out_specs=pl.BlockSpec((1, n, c), lambda i: (i, 0, 0)),
        out_shape=jax.ShapeDtypeStruct((b, n, c), jnp.float32),
    )(x)


def _fused_topk_kernel(fx_ref, fyn_ref, idx_ref, val_ref, *, ny, k):
    fx = fx_ref[0]
    fyn = fyn_ref[0]

    nx = jnp.sqrt(jnp.sum(fx * fx, axis=-1, keepdims=True))
    fxn = fx / jnp.maximum(nx, 1e-12)

    simt = jax.lax.dot_general(
        fyn, fxn, (((1,), (1,)), ((), ())),
        preferred_element_type=jnp.float32,
    )

    r = simt.shape[1]
    cw = _CW
    nch = ny // cw
    neg = jnp.finfo(jnp.float32).min
    big = jnp.int32(1 << 30)

    iota_ch = jax.lax.broadcasted_iota(jnp.int32, (nch, r), 0)
    iota_w = jax.lax.broadcasted_iota(jnp.int32, (cw, r), 0)

    slices = [simt[c * cw:(c + 1) * cw, :] for c in range(nch)]

    cms = []
    jms = []
    for s in slices:
        mx = jnp.max(s, axis=0, keepdims=True)
        cms.append(mx)
        jms.append(jnp.min(jnp.where(s == mx, iota_w, cw),
                           axis=0, keepdims=True))
    cm = jnp.concatenate(cms, axis=0)
    jm = jnp.concatenate(jms, axis=0)

    vals = []
    idxs = []
    for t in range(k):
        m = jnp.max(cm, axis=0, keepdims=True)
        g = jnp.min(jnp.where(cm == m, iota_ch * cw + jm, big),
                    axis=0, keepdims=True)
        vals.append(m)
        idxs.append(g)
        if t == k - 1:
            break
        c = jax.lax.shift_right_logical(g, cw.bit_length() - 1)
        p = jax.lax.bitwise_and(g, cw - 1)
        colv = slices[0]
        for j in range(1, nch):
            colv = jnp.where(c == j, slices[j], colv)
        keep = (colv < m) | ((colv == m) & (iota_w > p))
        rem = jnp.where(keep, colv, neg)
        nm = jnp.max(rem, axis=0, keepdims=True)
        npos = jnp.min(jnp.where(rem == nm, iota_w, cw),
                       axis=0, keepdims=True)
        cm = jnp.where(iota_ch == c, nm, cm)
        jm = jnp.where(iota_ch == c, npos, jm)

    vals.append(jnp.full((1, r), neg, jnp.float32))
    idxs.append(jnp.zeros((1, r), jnp.int32))
    v = jnp.concatenate(vals, axis=0)
    i = jnp.concatenate(idxs, axis=0)

    vt = v / jnp.float32(_TAU)
    e = jnp.exp(vt - vt[:1, :])
    sm = e / jnp.sum(e[:_K, :], axis=0, keepdims=True)

    idx_ref[0] = i
    val_ref[0] = sm


def kernel(feat_x, feat_y):
    b, nx, c = feat_x.shape
    ny = feat_y.shape[1]
    r = 256
    grid = (b, nx // r)

    fyn = _l2norm(feat_y)
    body = functools.partial(_fused_topk_kernel, ny=ny, k=_K)

    idx, val = pl.pallas_call(
        body,
        grid=grid,
        in_specs=[
            pl.BlockSpec((1, r, c), lambda bi, i: (bi, i, 0)),
            pl.BlockSpec((1, ny, c), lambda bi, i: (bi, 0, 0)),
        ],
        out_specs=[
            pl.BlockSpec((1, _KP, r), lambda bi, i: (bi, 0, i)),
            pl.BlockSpec((1, _KP, r), lambda bi, i: (bi, 0, i)),
        ],
        out_shape=[
            jax.ShapeDtypeStruct((b, _KP, nx), jnp.int32),
            jax.ShapeDtypeStruct((b, _KP, nx), jnp.float32),
        ],
        compiler_params=pltpu.CompilerParams(
            dimension_semantics=("parallel", "parallel"),
        ),
    )(feat_x, fyn)
    idx = jnp.transpose(idx[:, :_K, :], (0, 2, 1))
    val = jnp.transpose(val[:, :_K, :], (0, 2, 1))
    return idx, val
